# Initial kernel scaffold; baseline (speedup 1.0000x reference)
import jax
import jax.numpy as jnp
from jax.experimental import pallas as pl


def kernel(edge_index, edge_values, features):
    raise NotImplementedError("write your pallas kernel here")


# speedup vs baseline: 4.0945x; 4.0945x over previous
import dataclasses
import functools

import jax
import jax.numpy as jnp
from jax import lax
from jax.experimental import pallas as pl
from jax.experimental.pallas import tpu as pltpu
from jax.experimental.pallas import tpu_sc as plsc

N_NODES = 10000
N_EDGES = 320000
D = 128
LANES = 16

NC, NS = 2, 16
NW = NC * NS
EDGES_PER_W = N_EDGES // NW
K = 80
CHUNKS = EDGES_PER_W // K
ROWS_PER_TILE = N_NODES // NS

_mesh = plsc.VectorSubcoreMesh(core_axis_name="c", subcore_axis_name="s")

_cp = pltpu.CompilerParams()
if "needs_layout_passes" in pltpu.CompilerParams.__dataclass_fields__:
    _cp = dataclasses.replace(_cp, needs_layout_passes=False)


@functools.partial(
    pl.kernel,
    out_type=jax.ShapeDtypeStruct((NC, N_NODES, D), jnp.float32),
    mesh=_mesh,
    compiler_params=_cp,
    scratch_types=[
        pltpu.VMEM((K,), jnp.int32),
        pltpu.VMEM((K,), jnp.int32),
        pltpu.VMEM((K,), jnp.float32),
        pltpu.VMEM((K, D), jnp.float32),
        pltpu.VMEM_SHARED((N_NODES, D), jnp.float32),
    ],
)
def _spmm_sc(row_hbm, col_hbm, val_hbm, feat_hbm, out_hbm, col_v, row_v,
             val_v, gbuf, acc):
    cid = lax.axis_index("c")
    sid = lax.axis_index("s")
    wid = sid * NC + cid
    base = wid * EDGES_PER_W

    zero = jnp.zeros((LANES,), jnp.float32)

    @pl.loop(0, K)
    def _(j):
        for t in range(D // LANES):
            gbuf[j, pl.ds(t * LANES, LANES)] = zero

    n_row_chunks = N_NODES // K

    @pl.loop(sid, n_row_chunks, step=NS)
    def _(ci):
        pltpu.sync_copy(gbuf, acc.at[pl.ds(ci * K, K)])

    plsc.subcore_barrier()

    @pl.loop(0, CHUNKS)
    def _(ci):
        off = base + ci * K
        pltpu.sync_copy(col_hbm.at[pl.ds(off, K)], col_v)
        pltpu.sync_copy(row_hbm.at[pl.ds(off, K)], row_v)
        pltpu.sync_copy(val_hbm.at[pl.ds(off, K)], val_v)

        pltpu.sync_copy(feat_hbm.at[col_v], gbuf)

        @pl.loop(0, K)
        def _(j):
            vv = plsc.load_gather(val_v, [jnp.full((LANES,), j, jnp.int32)])
            for t in range(D // LANES):
                sl = pl.ds(t * LANES, LANES)
                gbuf[j, sl] = gbuf[j, sl] * vv

        pltpu.sync_copy(gbuf, acc.at[row_v], add=True)

    plsc.subcore_barrier()

    @pl.loop(sid, n_row_chunks, step=NS)
    def _(ci):
        pltpu.sync_copy(acc.at[pl.ds(ci * K, K)],
                        out_hbm.at[cid, pl.ds(ci * K, K)])


def _combine_body(p_ref, o_ref):
    o_ref[...] = p_ref[0] + p_ref[1]


def kernel(edge_index, edge_values, features):
    partials = _spmm_sc(edge_index[0], edge_index[1], edge_values, features)
    out = pl.pallas_call(
        _combine_body,
        out_shape=jax.ShapeDtypeStruct((N_NODES, D), jnp.float32),
        grid=(5,),
        in_specs=[pl.BlockSpec((2, N_NODES // 5, D), lambda i: (0, i, 0))],
        out_specs=pl.BlockSpec((N_NODES // 5, D), lambda i: (i, 0)),
    )(partials)
    return out
